# baseline (speedup 1.0000x reference)
import jax
import jax.numpy as jnp
from jax import lax
from jax.experimental import pallas as pl
from jax.experimental.pallas import tpu as pltpu
from jax.experimental.pallas import tpu_sc as plsc

_info = plsc.get_sparse_core_info()
_NC, _NS = _info.num_cores, _info.num_subcores
_NW = _NC * _NS

_NBUF = 4
_CHUNK = 800

_mesh = plsc.VectorSubcoreMesh(core_axis_name="c", subcore_axis_name="s")


def _gather_body(nchunk):
    c, nb = _CHUNK, _NBUF
    k = nb - 1

    def body(table, idx_hbm, out, *scratch):
        ibufs = scratch[0:nb]
        rbufs = scratch[nb:2 * nb]
        isems = scratch[2 * nb:3 * nb]
        gsems = scratch[3 * nb:4 * nb]
        wsems = scratch[4 * nb:5 * nb]

        wid = lax.axis_index("s") * _NC + lax.axis_index("c")
        wbase = wid * (nchunk * c)

        def off(g):
            return wbase + g * c

        def idx_start(g, j):
            pltpu.async_copy(idx_hbm.at[pl.ds(off(g), c)], ibufs[j], isems[j])

        def idx_wait(j):
            pltpu.make_async_copy(idx_hbm.at[pl.ds(wbase, c)], ibufs[j], isems[j]).wait()

        def gat_start(j):
            pltpu.async_copy(table.at[ibufs[j]], rbufs[j], gsems[j])

        def gat_wait(j):
            pltpu.make_async_copy(table.at[ibufs[j]], rbufs[j], gsems[j]).wait()

        def wb_start(g, j):
            pltpu.async_copy(rbufs[j], out.at[pl.ds(off(g), c)], wsems[j])

        def wb_wait(j):
            pltpu.make_async_copy(rbufs[j], out.at[pl.ds(wbase, c)], wsems[j]).wait()

        def step(g, j, wait_prev, start_g, start_i):
            gat_wait(j)
            wb_start(g, j)
            if wait_prev:
                wb_wait((j - 1) % nb)
            if start_g:
                jj = (j + k) % nb
                idx_wait(jj)
                gat_start(jj)
            if start_i:
                idx_start(g + nb, j)

        for g in range(nb):
            idx_start(g, g)
        for g in range(k):
            idx_wait(g)
            gat_start(g)

        for j in range(nb):
            step(j, j, j >= 1, j + k < nchunk, j + nb < nchunk)

        nblocks = nchunk // nb

        def block(i, carry):
            g0 = i * nb
            for j in range(nb):
                step(g0 + j, j, True, True, True)
            return carry

        lax.fori_loop(1, nblocks - 1, block, 0)

        for j in range(nb):
            g = nchunk - nb + j
            step(g, j, True, g + k < nchunk, g + nb < nchunk)

        wb_wait(nb - 1)

    return body


def _relayout_body(hist, d, batch):
    slab = batch // _NW
    flen = slab * d
    nk = slab // 16

    def body(rows, out, vi0, vi1, vo0, vo1, is0, is1, os0, os1):
        wid = lax.axis_index("s") * _NC + lax.axis_index("c")
        bbase = wid * slab
        iota32 = lax.iota(jnp.int32, 16) * d

        vis, vos = (vi0, vi1), (vo0, vo1)
        isems, osems = (is0, is1), (os0, os1)

        def in_start(h, p):
            pltpu.async_copy(
                rows.at[pl.ds((h * batch + bbase) * d, flen)], vis[p], isems[p])

        def in_wait(p):
            pltpu.make_async_copy(
                rows.at[pl.ds(0, flen)], vis[p], isems[p]).wait()

        def out_start(h, p):
            pltpu.async_copy(
                vos[p], out.at[pl.ds(h, 1), :, pl.ds(bbase, slab)], osems[p])

        def out_wait(p):
            pltpu.make_async_copy(
                vos[p], out.at[pl.ds(0, 1), :, pl.ds(bbase, slab)], osems[p]).wait()

        def transpose(p):
            vin, vout = vis[p], vos[p]

            @plsc.parallel_loop(0, nk, unroll=4)
            def kbody(kk):
                b0 = kk * 16
                base = b0 * d
                for dd in range(d):
                    v = plsc.load_gather(vin, [base + dd + iota32])
                    vout[0, dd, pl.ds(b0, 16)] = v

        def step(h, p, start_next_in, wait_out_prev):
            in_wait(p)
            if start_next_in:
                in_start(h + 1, 1 - p)
            if wait_out_prev:
                out_wait(p)
            transpose(p)
            out_start(h, p)

        in_start(0, 0)
        step(0, 0, True, False)
        step(1, 1, True, False)

        def pair(i, carry):
            step(2 * i, 0, True, True)
            step(2 * i + 1, 1, True, True)
            return carry

        lax.fori_loop(1, hist // 2 - 1, pair, 0)

        step(hist - 2, 0, True, True)
        step(hist - 1, 1, False, True)

        out_wait(0)
        out_wait(1)

    return body


def kernel(token_ids, embedding_table):
    batch, hist = token_ids.shape
    n_emb, d = embedding_table.shape
    b = batch * hist
    assert b % (_NW * _CHUNK) == 0
    nchunk = b // (_NW * _CHUNK)
    assert nchunk % _NBUF == 0 and nchunk // _NBUF >= 3
    slab = batch // _NW
    assert slab % 16 == 0 and hist % 2 == 0 and hist >= 6

    ids_flat = jnp.transpose(token_ids).reshape(b).astype(jnp.int32)

    rows = pl.kernel(
        _gather_body(nchunk),
        out_type=jax.ShapeDtypeStruct((b, d), jnp.float32),
        mesh=_mesh,
        compiler_params=pltpu.CompilerParams(use_tc_tiling_on_sc=False),
        scratch_types=(
            [pltpu.VMEM((_CHUNK,), jnp.int32) for _ in range(_NBUF)]
            + [pltpu.VMEM((_CHUNK, d), jnp.float32) for _ in range(_NBUF)]
            + [pltpu.SemaphoreType.DMA for _ in range(3 * _NBUF)]
        ),
    )(embedding_table, ids_flat)
    rows_flat = rows.reshape(b * d)

    o2 = pl.kernel(
        _relayout_body(hist, d, batch),
        out_type=jax.ShapeDtypeStruct((hist, d, batch), jnp.float32),
        mesh=_mesh,
        compiler_params=pltpu.CompilerParams(
            use_tc_tiling_on_sc=True, needs_layout_passes=False),
        scratch_types=[
            pltpu.VMEM((slab * d,), jnp.float32),
            pltpu.VMEM((slab * d,), jnp.float32),
            pltpu.VMEM((1, d, slab), jnp.float32),
            pltpu.VMEM((1, d, slab), jnp.float32),
            pltpu.SemaphoreType.DMA,
            pltpu.SemaphoreType.DMA,
            pltpu.SemaphoreType.DMA,
            pltpu.SemaphoreType.DMA,
        ],
    )(rows_flat)
    return jnp.transpose(o2, (2, 0, 1))


# speedup vs baseline: 1.1283x; 1.1283x over previous
import jax
import jax.numpy as jnp
from jax import lax
from jax.experimental import pallas as pl
from jax.experimental.pallas import tpu as pltpu
from jax.experimental.pallas import tpu_sc as plsc

_info = plsc.get_sparse_core_info()
_NC, _NS = _info.num_cores, _info.num_subcores
_NW = _NC * _NS

_NBUF = 4
_CHUNK = 800

_mesh = plsc.VectorSubcoreMesh(core_axis_name="c", subcore_axis_name="s")


def _gather_body(nchunk):
    c, nb = _CHUNK, _NBUF
    k = nb - 1

    def body(table, idx_hbm, out, *scratch):
        ibufs = scratch[0:nb]
        rbufs = scratch[nb:2 * nb]
        isems = scratch[2 * nb:3 * nb]
        gsems = scratch[3 * nb:4 * nb]
        wsems = scratch[4 * nb:5 * nb]

        wid = lax.axis_index("s") * _NC + lax.axis_index("c")
        wbase = wid * (nchunk * c)

        def off(g):
            return wbase + g * c

        def idx_start(g, j):
            pltpu.async_copy(idx_hbm.at[pl.ds(off(g), c)], ibufs[j], isems[j])

        def idx_wait(j):
            pltpu.make_async_copy(idx_hbm.at[pl.ds(wbase, c)], ibufs[j], isems[j]).wait()

        def gat_start(j):
            pltpu.async_copy(table.at[ibufs[j]], rbufs[j], gsems[j])

        def gat_wait(j):
            pltpu.make_async_copy(table.at[ibufs[j]], rbufs[j], gsems[j]).wait()

        def wb_start(g, j):
            pltpu.async_copy(rbufs[j], out.at[pl.ds(off(g), c)], wsems[j])

        def wb_wait(j):
            pltpu.make_async_copy(rbufs[j], out.at[pl.ds(wbase, c)], wsems[j]).wait()

        def step(g, j, wait_prev, start_g, start_i):
            gat_wait(j)
            wb_start(g, j)
            if wait_prev:
                wb_wait((j - 1) % nb)
            if start_g:
                jj = (j + k) % nb
                idx_wait(jj)
                gat_start(jj)
            if start_i:
                idx_start(g + nb, j)

        for g in range(nb):
            idx_start(g, g)
        for g in range(k):
            idx_wait(g)
            gat_start(g)

        for j in range(nb):
            step(j, j, j >= 1, j + k < nchunk, j + nb < nchunk)

        nblocks = nchunk // nb

        def block(i, carry):
            g0 = i * nb
            for j in range(nb):
                step(g0 + j, j, True, True, True)
            return carry

        lax.fori_loop(1, nblocks - 1, block, 0)

        for j in range(nb):
            g = nchunk - nb + j
            step(g, j, True, g + k < nchunk, g + nb < nchunk)

        wb_wait(nb - 1)

    return body


def _relayout_body(hist, d, batch):
    slab = batch // _NW
    flen = slab * d
    nk = slab // 16

    def body(rows, out, vi0, vi1, vo0, vo1, is0, is1, os0, os1):
        wid = lax.axis_index("s") * _NC + lax.axis_index("c")
        bbase = wid * slab
        iota32 = lax.iota(jnp.int32, 16) * d

        vis, vos = (vi0, vi1), (vo0, vo1)
        isems, osems = (is0, is1), (os0, os1)

        def in_start(h, p):
            pltpu.async_copy(
                rows.at[pl.ds((h * batch + bbase) * d, flen)], vis[p], isems[p])

        def in_wait(p):
            pltpu.make_async_copy(
                rows.at[pl.ds(0, flen)], vis[p], isems[p]).wait()

        def out_start(h, p):
            pltpu.async_copy(
                vos[p], out.at[pl.ds(h, 1), :, pl.ds(bbase, slab)], osems[p])

        def out_wait(p):
            pltpu.make_async_copy(
                vos[p], out.at[pl.ds(0, 1), :, pl.ds(bbase, slab)], osems[p]).wait()

        def transpose(p):
            vin, vout = vis[p], vos[p]

            @plsc.parallel_loop(0, d, unroll=2)
            def dbody(dd):
                ivec = dd + iota32
                for kk in range(nk):
                    v = plsc.load_gather(vin, [ivec + (kk * 16 * d)])
                    vout[0, dd, pl.ds(kk * 16, 16)] = v

        def step(h, p, start_next_in, wait_out_prev):
            in_wait(p)
            if start_next_in:
                in_start(h + 1, 1 - p)
            if wait_out_prev:
                out_wait(p)
            transpose(p)
            out_start(h, p)

        in_start(0, 0)
        step(0, 0, True, False)
        step(1, 1, True, False)

        def pair(i, carry):
            step(2 * i, 0, True, True)
            step(2 * i + 1, 1, True, True)
            return carry

        lax.fori_loop(1, hist // 2 - 1, pair, 0)

        step(hist - 2, 0, True, True)
        step(hist - 1, 1, False, True)

        out_wait(0)
        out_wait(1)

    return body


def kernel(token_ids, embedding_table):
    batch, hist = token_ids.shape
    n_emb, d = embedding_table.shape
    b = batch * hist
    assert b % (_NW * _CHUNK) == 0
    nchunk = b // (_NW * _CHUNK)
    assert nchunk % _NBUF == 0 and nchunk // _NBUF >= 3
    slab = batch // _NW
    assert slab % 16 == 0 and hist % 2 == 0 and hist >= 6

    ids_flat = jnp.transpose(token_ids).reshape(b).astype(jnp.int32)

    rows = pl.kernel(
        _gather_body(nchunk),
        out_type=jax.ShapeDtypeStruct((b, d), jnp.float32),
        mesh=_mesh,
        compiler_params=pltpu.CompilerParams(use_tc_tiling_on_sc=False),
        scratch_types=(
            [pltpu.VMEM((_CHUNK,), jnp.int32) for _ in range(_NBUF)]
            + [pltpu.VMEM((_CHUNK, d), jnp.float32) for _ in range(_NBUF)]
            + [pltpu.SemaphoreType.DMA for _ in range(3 * _NBUF)]
        ),
    )(embedding_table, ids_flat)
    rows_flat = rows.reshape(b * d)

    o2 = pl.kernel(
        _relayout_body(hist, d, batch),
        out_type=jax.ShapeDtypeStruct((hist, d, batch), jnp.float32),
        mesh=_mesh,
        compiler_params=pltpu.CompilerParams(
            use_tc_tiling_on_sc=True, needs_layout_passes=False),
        scratch_types=[
            pltpu.VMEM((slab * d,), jnp.float32),
            pltpu.VMEM((slab * d,), jnp.float32),
            pltpu.VMEM((1, d, slab), jnp.float32),
            pltpu.VMEM((1, d, slab), jnp.float32),
            pltpu.SemaphoreType.DMA,
            pltpu.SemaphoreType.DMA,
            pltpu.SemaphoreType.DMA,
            pltpu.SemaphoreType.DMA,
        ],
    )(rows_flat)
    return jnp.transpose(o2, (2, 0, 1))


# speedup vs baseline: 1.1295x; 1.0011x over previous
import jax
import jax.numpy as jnp
from jax import lax
from jax.experimental import pallas as pl
from jax.experimental.pallas import tpu as pltpu
from jax.experimental.pallas import tpu_sc as plsc

_info = plsc.get_sparse_core_info()
_NC, _NS = _info.num_cores, _info.num_subcores
_NW = _NC * _NS

_NBUF = 4
_CHUNK = 800

_mesh = plsc.VectorSubcoreMesh(core_axis_name="c", subcore_axis_name="s")


def _gather_body(nchunk):
    c, nb = _CHUNK, _NBUF
    k = nb - 1

    def body(table, idx_hbm, out, *scratch):
        ibufs = scratch[0:nb]
        rbufs = scratch[nb:2 * nb]
        isems = scratch[2 * nb:3 * nb]
        gsems = scratch[3 * nb:4 * nb]
        wsems = scratch[4 * nb:5 * nb]

        wid = lax.axis_index("s") * _NC + lax.axis_index("c")
        wbase = wid * (nchunk * c)

        def off(g):
            return wbase + g * c

        def idx_start(g, j):
            pltpu.async_copy(idx_hbm.at[pl.ds(off(g), c)], ibufs[j], isems[j])

        def idx_wait(j):
            pltpu.make_async_copy(idx_hbm.at[pl.ds(wbase, c)], ibufs[j], isems[j]).wait()

        def gat_start(j):
            pltpu.async_copy(table.at[ibufs[j]], rbufs[j], gsems[j])

        def gat_wait(j):
            pltpu.make_async_copy(table.at[ibufs[j]], rbufs[j], gsems[j]).wait()

        def wb_start(g, j):
            pltpu.async_copy(rbufs[j], out.at[pl.ds(off(g), c)], wsems[j])

        def wb_wait(j):
            pltpu.make_async_copy(rbufs[j], out.at[pl.ds(wbase, c)], wsems[j]).wait()

        def step(g, j, wait_prev, start_g, start_i):
            gat_wait(j)
            wb_start(g, j)
            if wait_prev:
                wb_wait((j - 1) % nb)
            if start_g:
                jj = (j + k) % nb
                idx_wait(jj)
                gat_start(jj)
            if start_i:
                idx_start(g + nb, j)

        for g in range(nb):
            idx_start(g, g)
        for g in range(k):
            idx_wait(g)
            gat_start(g)

        for j in range(nb):
            step(j, j, j >= 1, j + k < nchunk, j + nb < nchunk)

        nblocks = nchunk // nb

        def block(i, carry):
            g0 = i * nb
            for j in range(nb):
                step(g0 + j, j, True, True, True)
            return carry

        lax.fori_loop(1, nblocks - 1, block, 0)

        for j in range(nb):
            g = nchunk - nb + j
            step(g, j, True, g + k < nchunk, g + nb < nchunk)

        wb_wait(nb - 1)

    return body


def _relayout_body(hist, d, batch):
    slab = batch // _NW
    flen = slab * d
    nk = slab // 16

    def body(rows, out, vi0, vi1, vo0, vo1, is0, is1, os0, os1):
        wid = lax.axis_index("s") * _NC + lax.axis_index("c")
        bbase = wid * slab
        iota32 = lax.iota(jnp.int32, 16) * d

        vis, vos = (vi0, vi1), (vo0, vo1)
        isems, osems = (is0, is1), (os0, os1)

        def in_start(h, p):
            pltpu.async_copy(
                rows.at[pl.ds((h * batch + bbase) * d, flen)], vis[p], isems[p])

        def in_wait(p):
            pltpu.make_async_copy(
                rows.at[pl.ds(0, flen)], vis[p], isems[p]).wait()

        def out_start(h, p):
            pltpu.async_copy(
                vos[p], out.at[pl.ds(h, 1), :, pl.ds(bbase, slab)], osems[p])

        def out_wait(p):
            pltpu.make_async_copy(
                vos[p], out.at[pl.ds(0, 1), :, pl.ds(bbase, slab)], osems[p]).wait()

        def transpose(p):
            vin, vout = vis[p], vos[p]

            @plsc.parallel_loop(0, d, unroll=4)
            def dbody(dd):
                ivec = dd + iota32
                for kk in range(nk):
                    v = plsc.load_gather(vin, [ivec + (kk * 16 * d)])
                    vout[0, dd, pl.ds(kk * 16, 16)] = v

        def step(h, p, start_next_in, wait_out_prev):
            in_wait(p)
            if start_next_in:
                in_start(h + 1, 1 - p)
            if wait_out_prev:
                out_wait(p)
            transpose(p)
            out_start(h, p)

        in_start(0, 0)
        step(0, 0, True, False)
        step(1, 1, True, False)

        def pair(i, carry):
            step(2 * i, 0, True, True)
            step(2 * i + 1, 1, True, True)
            return carry

        lax.fori_loop(1, hist // 2 - 1, pair, 0)

        step(hist - 2, 0, True, True)
        step(hist - 1, 1, False, True)

        out_wait(0)
        out_wait(1)

    return body


def kernel(token_ids, embedding_table):
    batch, hist = token_ids.shape
    n_emb, d = embedding_table.shape
    b = batch * hist
    assert b % (_NW * _CHUNK) == 0
    nchunk = b // (_NW * _CHUNK)
    assert nchunk % _NBUF == 0 and nchunk // _NBUF >= 3
    slab = batch // _NW
    assert slab % 16 == 0 and hist % 2 == 0 and hist >= 6

    ids_flat = jnp.transpose(token_ids).reshape(b).astype(jnp.int32)

    rows = pl.kernel(
        _gather_body(nchunk),
        out_type=jax.ShapeDtypeStruct((b, d), jnp.float32),
        mesh=_mesh,
        compiler_params=pltpu.CompilerParams(use_tc_tiling_on_sc=False),
        scratch_types=(
            [pltpu.VMEM((_CHUNK,), jnp.int32) for _ in range(_NBUF)]
            + [pltpu.VMEM((_CHUNK, d), jnp.float32) for _ in range(_NBUF)]
            + [pltpu.SemaphoreType.DMA for _ in range(3 * _NBUF)]
        ),
    )(embedding_table, ids_flat)
    rows_flat = rows.reshape(b * d)

    o2 = pl.kernel(
        _relayout_body(hist, d, batch),
        out_type=jax.ShapeDtypeStruct((hist, d, batch), jnp.float32),
        mesh=_mesh,
        compiler_params=pltpu.CompilerParams(
            use_tc_tiling_on_sc=True, needs_layout_passes=False),
        scratch_types=[
            pltpu.VMEM((slab * d,), jnp.float32),
            pltpu.VMEM((slab * d,), jnp.float32),
            pltpu.VMEM((1, d, slab), jnp.float32),
            pltpu.VMEM((1, d, slab), jnp.float32),
            pltpu.SemaphoreType.DMA,
            pltpu.SemaphoreType.DMA,
            pltpu.SemaphoreType.DMA,
            pltpu.SemaphoreType.DMA,
        ],
    )(rows_flat)
    return jnp.transpose(o2, (2, 0, 1))
